# baseline (speedup 1.0000x reference)
import functools

import jax
import jax.numpy as jnp
from jax import lax
from jax.experimental import pallas as pl
from jax.experimental.pallas import tpu as pltpu
from jax.experimental.pallas import tpu_sc as plsc

N_NODES = 10000
N_EDGES = 320000
D_IN = 128
HEADS1 = 3
HID = 256

NC, NS, NW = 2, 16, 32
MBLK = 1000
EB = 80
NPT = N_NODES // NS

_mesh = plsc.VectorSubcoreMesh(core_axis_name="c", subcore_axis_name="s")


def _iota16():
    return jnp.arange(16, dtype=jnp.int32)


def _make_logits(heads):
    epw = N_EDGES // NW
    nb = epw // EB
    scale = 1.0 / (HID ** 0.5)

    @functools.partial(
        pl.kernel, mesh=_mesh, name=f"k1_logits_h{heads}",
        compiler_params=pltpu.CompilerParams(use_tc_tiling_on_sc=False, needs_layout_passes=False),
        out_type=jax.ShapeDtypeStruct((heads * N_EDGES,), jnp.float32),
        scratch_types=[
            pltpu.VMEM((2, EB), jnp.int32),
            pltpu.VMEM((2, EB), jnp.int32),
            pltpu.VMEM((2 * EB, HID), jnp.float32),
            pltpu.VMEM((2 * EB, HID), jnp.float32),
            pltpu.VMEM((EB,), jnp.float32),
            pltpu.VMEM((16, 16), jnp.float32),
            pltpu.SemaphoreType.DMA((2,)),
            pltpu.SemaphoreType.DMA((2,)),
        ])
    def k(q_hbm, k_hbm, src_hbm, dst_hbm, out_hbm,
          dstv2, srcv2, qrows2, krows2, lstage, tmp, isem, gsem):
        wid = lax.axis_index("s") * NC + lax.axis_index("c")
        base = wid * epw

        def issue_ids(b, p):
            e0 = base + b * EB
            pltpu.async_copy(dst_hbm.at[pl.ds(e0, EB)], dstv2.at[p], isem.at[p])
            pltpu.async_copy(src_hbm.at[pl.ds(e0, EB)], srcv2.at[p], isem.at[p])

        def drain_ids(p):
            pltpu.make_async_copy(dst_hbm.at[pl.ds(0, EB)], dstv2.at[p], isem.at[p]).wait()
            pltpu.make_async_copy(src_hbm.at[pl.ds(0, EB)], srcv2.at[p], isem.at[p]).wait()

        def issue_g(pg, pb, h):
            pltpu.async_copy(q_hbm.at[h].at[dstv2.at[pb]],
                             qrows2.at[pl.ds(pg * EB, EB)], gsem.at[pg])
            pltpu.async_copy(k_hbm.at[h].at[srcv2.at[pb]],
                             krows2.at[pl.ds(pg * EB, EB)], gsem.at[pg])

        def drain_g(pg, pb, h):
            pltpu.make_async_copy(q_hbm.at[h].at[dstv2.at[pb]],
                                  qrows2.at[pl.ds(pg * EB, EB)], gsem.at[pg]).wait()
            pltpu.make_async_copy(k_hbm.at[h].at[srcv2.at[pb]],
                                  krows2.at[pl.ds(pg * EB, EB)], gsem.at[pg]).wait()

        def compute(b, h, pg):
            e0 = base + b * EB
            rbase = pg * EB

            def group(g, cc):
                @plsc.parallel_loop(0, 16, unroll=4)
                def edge(jj):
                    j = rbase + g * 16 + jj
                    sl0 = pl.ds(0, 16)
                    pv = qrows2[j, sl0] * krows2[j, sl0]
                    for c16 in range(1, HID // 16):
                        sl = pl.ds(c16 * 16, 16)
                        pv = pv + qrows2[j, sl] * krows2[j, sl]
                    plsc.store_scatter(
                        tmp, [_iota16(), jnp.full((16,), 0, jnp.int32) + jj], pv)
                t8 = [tmp[r, :] + tmp[r + 8, :] for r in range(8)]
                t4 = [t8[r] + t8[r + 4] for r in range(4)]
                t2 = [t4[r] + t4[r + 2] for r in range(2)]
                tot = t2[0] + t2[1]
                lstage[pl.ds(g * 16, 16)] = tot * scale
                return cc

            lax.fori_loop(0, EB // 16, group, 0)
            pltpu.sync_copy(lstage, out_hbm.at[pl.ds(h * N_EDGES + e0, EB)])

        issue_ids(0, 0)
        if nb > 1:
            issue_ids(1, 1)
        drain_ids(0)
        issue_g(0, 0, 0)

        def batch(b, carry):
            pb = lax.rem(b, 2)
            pbn = 1 - pb
            for h in range(heads):
                pg = lax.rem(b + h, 2)
                pgn = 1 - pg
                if h + 1 < heads:
                    issue_g(pgn, pb, h + 1)
                else:
                    @pl.when(b + 1 < nb)
                    def _():
                        drain_ids(pbn)
                        issue_g(pgn, pbn, 0)
                drain_g(pg, pb, h)
                if h == heads - 1:
                    @pl.when(b + 2 < nb)
                    def _():
                        issue_ids(b + 2, pb)
                compute(b, h, pg)
            return carry

        lax.fori_loop(0, nb, batch, 0)

    return k


def _gmax_body(l_ref, m_ref):
    i = pl.program_id(0)
    h = l_ref.shape[0]
    bm = jnp.max(l_ref[...], axis=1, keepdims=True)
    bm = jnp.broadcast_to(bm, (h, 128))
    bm = jnp.concatenate([bm, jnp.zeros((8 - h, 128), jnp.float32)], axis=0)

    @pl.when(i == 0)
    def _():
        m_ref[...] = bm

    @pl.when(i != 0)
    def _():
        m_ref[...] = jnp.maximum(m_ref[...], bm)


def _gmax(logits_flat, heads):
    l2 = logits_flat.reshape(heads, N_EDGES)
    eb = 2560
    return pl.pallas_call(
        _gmax_body,
        grid=(N_EDGES // eb,),
        in_specs=[pl.BlockSpec((heads, eb), lambda i: (0, i))],
        out_specs=pl.BlockSpec((8, 128), lambda i: (0, 0)),
        out_shape=jax.ShapeDtypeStruct((8, 128), jnp.float32),
    )(l2).reshape(1024)


def _make_denom(heads):
    ept = N_EDGES // NW
    nb = ept // EB

    @functools.partial(
        pl.kernel, mesh=_mesh, name=f"k2_denom_h{heads}",
        compiler_params=pltpu.CompilerParams(use_tc_tiling_on_sc=False, needs_layout_passes=False),
        out_type=jax.ShapeDtypeStruct((NC * N_NODES, 16), jnp.float32),
        scratch_types=[
            pltpu.VMEM((EB,), jnp.int32),
            pltpu.VMEM((EB,), jnp.float32),
            pltpu.VMEM((EB, 16), jnp.float32),
            pltpu.VMEM((16,), jnp.float32),
            pltpu.VMEM_SHARED((N_NODES, 16), jnp.float32),
            pltpu.SemaphoreType.DMA,
        ])
    def k(l_hbm, m_hbm, dst_hbm, z_hbm, out_hbm,
          dstv, lbuf, estage, mbuf, dacc, sem):
        c = lax.axis_index("c")
        s = lax.axis_index("s")
        pltpu.sync_copy(z_hbm, dacc.at[pl.ds(s * NPT, NPT)])
        pltpu.sync_copy(z_hbm.at[pl.ds(0, EB)], estage)
        plsc.subcore_barrier()

        mvals = []
        for h in range(heads):
            pltpu.sync_copy(m_hbm.at[pl.ds(h * 128, 16)], mbuf)
            mvals.append(mbuf[...])

        base = (c * NS + s) * ept

        def batch(b, carry):
            e0 = base + b * EB
            pltpu.sync_copy(dst_hbm.at[pl.ds(e0, EB)], dstv)
            for h in range(heads):
                pltpu.sync_copy(l_hbm.at[pl.ds(h * N_EDGES + e0, EB)], lbuf)
                for g in range(EB // 16):
                    rows = _iota16() + (g * 16)
                    ev = jnp.exp(lbuf[pl.ds(g * 16, 16)] - mvals[h])
                    plsc.store_scatter(estage,
                                       [rows, jnp.full((16,), h, jnp.int32)], ev)
            pltpu.sync_copy(estage, dacc.at[dstv], add=True)
            return carry

        lax.fori_loop(0, nb, batch, 0)
        plsc.subcore_barrier()
        pltpu.sync_copy(dacc.at[pl.ds(s * NPT, NPT)],
                        out_hbm.at[pl.ds(c * N_NODES + s * NPT, NPT)])

    return k


def _dencomb_body(d_ref, r_ref):
    d = d_ref[0] + d_ref[1]
    r_ref[...] = 1.0 / (d + 1e-16)


def _dencomb(denoms):
    d3 = denoms.reshape(NC, N_NODES, 16)
    return pl.pallas_call(
        _dencomb_body,
        grid=(N_NODES // MBLK,),
        in_specs=[pl.BlockSpec((NC, MBLK, 16), lambda i: (0, i, 0))],
        out_specs=pl.BlockSpec((MBLK, 16), lambda i: (i, 0)),
        out_shape=jax.ShapeDtypeStruct((N_NODES, 16), jnp.float32),
    )(d3)


def _make_msg(heads):
    chunks = 2 * heads
    chs = chunks // NC
    ept = N_EDGES // NS
    nb = ept // EB

    @functools.partial(
        pl.kernel, mesh=_mesh, name=f"k3_msg_h{heads}",
        compiler_params=pltpu.CompilerParams(use_tc_tiling_on_sc=False, needs_layout_passes=False),
        out_type=jax.ShapeDtypeStruct((chunks * N_NODES, 128), jnp.float32),
        scratch_types=[
            pltpu.VMEM((2, EB), jnp.int32),
            pltpu.VMEM((2, EB), jnp.int32),
            pltpu.VMEM((2, EB), jnp.int32),
            pltpu.VMEM((2 * EB, 128), jnp.float32),
            pltpu.VMEM((EB,), jnp.float32),
            pltpu.VMEM((2 * EB, 16), jnp.float32),
            pltpu.VMEM((EB,), jnp.float32),
            pltpu.VMEM((16,), jnp.float32),
            pltpu.VMEM_SHARED((N_NODES, 128), jnp.float32),
            pltpu.SemaphoreType.DMA((2,)),
            pltpu.SemaphoreType.DMA((2,)),
        ])
    def k(v_hbm, src_hbm, dst_hbm, l_hbm, m_hbm, rd_hbm, z_hbm, out_hbm,
          srcv2, dstv2, vidx2, vrows2, lbuf, rdrows2, abuf, mbuf, acc,
          isem, gsem):
        c = lax.axis_index("c")
        s = lax.axis_index("s")
        base_e = s * ept

        for t in range(chs):
            ch = c * chs + t
            h = ch // 2
            pltpu.sync_copy(z_hbm, acc.at[pl.ds(s * NPT, NPT)])
            plsc.subcore_barrier()
            pltpu.sync_copy(m_hbm.at[pl.ds(h * 128, 16)], mbuf)
            mh = mbuf[...]
            hcols = jnp.full((16,), 0, jnp.int32) + h

            def issue_ids(b, p):
                e0 = base_e + b * EB
                pltpu.async_copy(src_hbm.at[pl.ds(e0, EB)], srcv2.at[p], isem.at[p])
                pltpu.async_copy(dst_hbm.at[pl.ds(e0, EB)], dstv2.at[p], isem.at[p])

            def drain_ids(p):
                pltpu.make_async_copy(src_hbm.at[pl.ds(0, EB)], srcv2.at[p], isem.at[p]).wait()
                pltpu.make_async_copy(dst_hbm.at[pl.ds(0, EB)], dstv2.at[p], isem.at[p]).wait()

            def issue_g(p):
                for g in range(EB // 16):
                    sl = pl.ds(g * 16, 16)
                    vidx2[p, sl] = srcv2[p, sl] + ch * N_NODES
                pltpu.async_copy(v_hbm.at[vidx2.at[p]],
                                 vrows2.at[pl.ds(p * EB, EB)], gsem.at[p])
                pltpu.async_copy(rd_hbm.at[dstv2.at[p]],
                                 rdrows2.at[pl.ds(p * EB, EB)], gsem.at[p])

            def drain_g(p):
                pltpu.make_async_copy(v_hbm.at[vidx2.at[p]],
                                      vrows2.at[pl.ds(p * EB, EB)], gsem.at[p]).wait()
                pltpu.make_async_copy(rd_hbm.at[dstv2.at[p]],
                                      rdrows2.at[pl.ds(p * EB, EB)], gsem.at[p]).wait()

            def compute(b, p):
                e0 = base_e + b * EB
                rbase = p * EB
                pltpu.sync_copy(l_hbm.at[pl.ds(h * N_EDGES + e0, EB)], lbuf)
                rrows0 = _iota16() + rbase
                for g in range(EB // 16):
                    rows = rrows0 + (g * 16)
                    ev = jnp.exp(lbuf[pl.ds(g * 16, 16)] - mh)
                    rd = plsc.load_gather(rdrows2, [rows, hcols])
                    abuf[pl.ds(g * 16, 16)] = ev * rd

                @plsc.parallel_loop(0, EB, unroll=8)
                def edge(j):
                    asp = plsc.load_gather(abuf, [jnp.full((16,), 0, jnp.int32) + j])
                    jr = rbase + j
                    for c8 in range(8):
                        sl = pl.ds(c8 * 16, 16)
                        vrows2[jr, sl] = vrows2[jr, sl] * asp
                pltpu.sync_copy(vrows2.at[pl.ds(p * EB, EB)],
                                acc.at[dstv2.at[p]], add=True)

            issue_ids(0, 0)
            issue_ids(1, 1)
            drain_ids(0)
            issue_g(0)

            def batch(b, carry):
                p = lax.rem(b, 2)
                pn = 1 - p

                @pl.when(b + 1 < nb)
                def _():
                    drain_ids(pn)
                    issue_g(pn)

                drain_g(p)
                compute(b, p)

                @pl.when(b + 2 < nb)
                def _():
                    issue_ids(b + 2, p)

                return carry

            lax.fori_loop(0, nb, batch, 0)
            plsc.subcore_barrier()
            pltpu.sync_copy(acc.at[pl.ds(s * NPT, NPT)],
                            out_hbm.at[pl.ds(ch * N_NODES + s * NPT, NPT)])

    return k


def _make_edgemlp():
    epw = N_EDGES // NW
    nb = epw // EB

    @functools.partial(
        pl.kernel, mesh=_mesh, name="k4_edgemlp",
        compiler_params=pltpu.CompilerParams(use_tc_tiling_on_sc=False, needs_layout_passes=False),
        out_type=jax.ShapeDtypeStruct((N_EDGES, HID), jnp.float32),
        scratch_types=[
            pltpu.VMEM((2, EB), jnp.int32),
            pltpu.VMEM((2, EB), jnp.int32),
            pltpu.VMEM((2 * EB, HID), jnp.float32),
            pltpu.VMEM((2 * EB, HID), jnp.float32),
            pltpu.SemaphoreType.DMA((2,)),
            pltpu.SemaphoreType.DMA((2,)),
        ])
    def k(a_hbm, b_hbm, src_hbm, dst_hbm, z_hbm,
          srcv2, dstv2, arows2, brows2, isem, gsem):
        wid = lax.axis_index("s") * NC + lax.axis_index("c")
        base = wid * epw

        def issue_ids(b, p):
            e0 = base + b * EB
            pltpu.async_copy(src_hbm.at[pl.ds(e0, EB)], srcv2.at[p], isem.at[p])
            pltpu.async_copy(dst_hbm.at[pl.ds(e0, EB)], dstv2.at[p], isem.at[p])

        def drain_ids(p):
            pltpu.make_async_copy(src_hbm.at[pl.ds(0, EB)], srcv2.at[p], isem.at[p]).wait()
            pltpu.make_async_copy(dst_hbm.at[pl.ds(0, EB)], dstv2.at[p], isem.at[p]).wait()

        def issue_g(p):
            pltpu.async_copy(a_hbm.at[srcv2.at[p]],
                             arows2.at[pl.ds(p * EB, EB)], gsem.at[p])
            pltpu.async_copy(b_hbm.at[dstv2.at[p]],
                             brows2.at[pl.ds(p * EB, EB)], gsem.at[p])

        def drain_g(p):
            pltpu.make_async_copy(a_hbm.at[srcv2.at[p]],
                                  arows2.at[pl.ds(p * EB, EB)], gsem.at[p]).wait()
            pltpu.make_async_copy(b_hbm.at[dstv2.at[p]],
                                  brows2.at[pl.ds(p * EB, EB)], gsem.at[p]).wait()

        def compute(b, p):
            e0 = base + b * EB
            rbase = p * EB

            @plsc.parallel_loop(0, EB, unroll=8)
            def edge(j):
                jr = rbase + j
                for c16 in range(HID // 16):
                    sl = pl.ds(c16 * 16, 16)
                    arows2[jr, sl] = jnp.maximum(
                        arows2[jr, sl] + brows2[jr, sl], 0.0)
            pltpu.sync_copy(arows2.at[pl.ds(p * EB, EB)], z_hbm.at[pl.ds(e0, EB)])

        issue_ids(0, 0)
        issue_ids(1, 1)
        drain_ids(0)
        issue_g(0)

        def batch(b, carry):
            p = lax.rem(b, 2)
            pn = 1 - p

            @pl.when(b + 1 < nb)
            def _():
                drain_ids(pn)
                issue_g(pn)

            drain_g(p)
            compute(b, p)

            @pl.when(b + 2 < nb)
            def _():
                issue_ids(b + 2, p)

            return carry

        lax.fori_loop(0, nb, batch, 0)

    return k


def _proj_body(x_ref, wq_ref, bq_ref, wk_ref, bk_ref, wv_ref, bv_ref,
               q_ref, k_ref, v_ref, *, heads):
    x = x_ref[...]
    for h in range(heads):
        sl = slice(h * HID, (h + 1) * HID)
        q_ref[h] = jnp.dot(x, wq_ref[:, sl], preferred_element_type=jnp.float32) + bq_ref[0, sl]
        k_ref[h] = jnp.dot(x, wk_ref[:, sl], preferred_element_type=jnp.float32) + bk_ref[0, sl]
    for ch in range(2 * heads):
        sl = slice(ch * 128, (ch + 1) * 128)
        v_ref[ch] = jnp.dot(x, wv_ref[:, sl], preferred_element_type=jnp.float32) + bv_ref[0, sl]


def _proj(x, wq, bq, wk, bk, wv, bv, heads):
    n, d = x.shape
    dout = heads * HID
    grid = (n // MBLK,)
    bspec_w = pl.BlockSpec((d, dout), lambda i: (0, 0))
    bspec_b = pl.BlockSpec((1, dout), lambda i: (0, 0))
    q, k, v = pl.pallas_call(
        functools.partial(_proj_body, heads=heads),
        grid=grid,
        in_specs=[pl.BlockSpec((MBLK, d), lambda i: (i, 0)),
                  bspec_w, bspec_b, bspec_w, bspec_b, bspec_w, bspec_b],
        out_specs=[pl.BlockSpec((heads, MBLK, HID), lambda i: (0, i, 0)),
                   pl.BlockSpec((heads, MBLK, HID), lambda i: (0, i, 0)),
                   pl.BlockSpec((2 * heads, MBLK, 128), lambda i: (0, i, 0))],
        out_shape=[jax.ShapeDtypeStruct((heads, n, HID), jnp.float32),
                   jax.ShapeDtypeStruct((heads, n, HID), jnp.float32),
                   jax.ShapeDtypeStruct((2 * heads, n, 128), jnp.float32)],
    )(x, wq, bq.reshape(1, dout), wk, bk.reshape(1, dout), wv, bv.reshape(1, dout))
    return q, k, v.reshape(2 * heads * n, 128)


def _bnstats_body(o_ref, s_ref, ss_ref):
    o = o_ref[...]
    s_ref[...] = jnp.sum(o, axis=1)[None]
    ss_ref[...] = jnp.sum(o * o, axis=1)[None]


def _bnapply_body(o_ref, s_ref, ss_ref, g_ref, be_ref, out_ref, *, n_total):
    ch = o_ref.shape[0]
    mean = jnp.sum(s_ref[...], axis=0) / n_total
    ex2 = jnp.sum(ss_ref[...], axis=0) / n_total
    var = ex2 - mean * mean
    scale = lax.rsqrt(var + 1e-5) * g_ref[...]
    shift = be_ref[...] - mean * scale
    o = o_ref[...] * scale[:, None, :] + shift[:, None, :]
    o = jnp.maximum(o, 0.0)
    out_ref[...] = o.transpose(1, 0, 2).reshape(o_ref.shape[1], ch * 128)


def _bn_relu(opre_flat, g, be, ch):
    opre = opre_flat.reshape(ch, N_NODES, 128)
    n = N_NODES
    grid = (n // MBLK,)
    nblk = n // MBLK
    s, ss = pl.pallas_call(
        _bnstats_body,
        grid=grid,
        in_specs=[pl.BlockSpec((ch, MBLK, 128), lambda i: (0, i, 0))],
        out_specs=[pl.BlockSpec((1, ch, 128), lambda i: (i, 0, 0)),
                   pl.BlockSpec((1, ch, 128), lambda i: (i, 0, 0))],
        out_shape=[jax.ShapeDtypeStruct((nblk, ch, 128), jnp.float32),
                   jax.ShapeDtypeStruct((nblk, ch, 128), jnp.float32)],
    )(opre)
    return pl.pallas_call(
        functools.partial(_bnapply_body, n_total=float(n)),
        grid=grid,
        in_specs=[pl.BlockSpec((ch, MBLK, 128), lambda i: (0, i, 0)),
                  pl.BlockSpec((nblk, ch, 128), lambda i: (0, 0, 0)),
                  pl.BlockSpec((nblk, ch, 128), lambda i: (0, 0, 0)),
                  pl.BlockSpec((ch, 128), lambda i: (0, 0)),
                  pl.BlockSpec((ch, 128), lambda i: (0, 0))],
        out_specs=pl.BlockSpec((MBLK, ch * 128), lambda i: (i, 0)),
        out_shape=jax.ShapeDtypeStruct((n, ch * 128), jnp.float32),
    )(opre, s, ss, g.reshape(ch, 128), be.reshape(ch, 128))


def _mm2_body(x_ref, wa_ref, wb_ref, bb_ref, a_ref, b_ref):
    x = x_ref[...]
    a_ref[...] = jnp.dot(x, wa_ref[...], preferred_element_type=jnp.float32)
    b_ref[...] = jnp.dot(x, wb_ref[...], preferred_element_type=jnp.float32) + bb_ref[...]


def _ab_matmul(x, wa, wb, bb):
    n, d = x.shape
    dout = wa.shape[1]
    sd = jax.ShapeDtypeStruct((n, dout), jnp.float32)
    return pl.pallas_call(
        _mm2_body,
        grid=(n // MBLK,),
        in_specs=[pl.BlockSpec((MBLK, d), lambda i: (i, 0)),
                  pl.BlockSpec((d, dout), lambda i: (0, 0)),
                  pl.BlockSpec((d, dout), lambda i: (0, 0)),
                  pl.BlockSpec((1, dout), lambda i: (0, 0))],
        out_specs=[pl.BlockSpec((MBLK, dout), lambda i: (i, 0)),
                   pl.BlockSpec((MBLK, dout), lambda i: (i, 0))],
        out_shape=[sd, sd],
    )(x, wa, wb, bb.reshape(1, dout))


def _mm_body(x_ref, w_ref, b_ref, o_ref):
    o_ref[...] = jnp.dot(x_ref[...], w_ref[...],
                         preferred_element_type=jnp.float32) + b_ref[...]


def _matmul(x, w, b, mblk=MBLK):
    n, d = x.shape
    dout = w.shape[1]
    return pl.pallas_call(
        _mm_body,
        grid=(n // mblk,),
        in_specs=[pl.BlockSpec((mblk, d), lambda i: (i, 0)),
                  pl.BlockSpec((d, dout), lambda i: (0, 0)),
                  pl.BlockSpec((1, dout), lambda i: (0, 0))],
        out_specs=pl.BlockSpec((mblk, dout), lambda i: (i, 0)),
        out_shape=jax.ShapeDtypeStruct((n, dout), jnp.float32),
    )(x, w, b.reshape(1, dout))


def _tgcn_layer(x, src, dst, wq, bq, wk, bk, wv, bv, g, be, heads,
                z16, z128):
    q2d, k2d, v2d = _proj(x, wq, bq, wk, bk, wv, bv, heads)
    logits = _make_logits(heads)(q2d, k2d, src, dst)
    m = _gmax(logits, heads)
    denoms = _make_denom(heads)(logits, m, dst, z16)
    rden = _dencomb(denoms)
    opre = _make_msg(heads)(v2d, src, dst, logits, m, rden, z128)
    return _bn_relu(opre, g, be, 2 * heads)


def kernel(x, edge_index, Wq1, bq1, Wk1, bk1, Wv1, bv1, g1, be1,
           Wq2, bq2, Wk2, bk2, Wv2, bv2, g2, be2, Wm1, bm1, Wm2, bm2):
    src = edge_index[0]
    dst = edge_index[1]
    z16 = jnp.zeros((NPT, 16), jnp.float32)
    z128 = jnp.zeros((NPT, 128), jnp.float32)

    h1 = _tgcn_layer(x, src, dst, Wq1, bq1, Wk1, bk1, Wv1, bv1, g1, be1,
                     HEADS1, z16, z128)
    h2 = _tgcn_layer(h1, src, dst, Wq2, bq2, Wk2, bk2, Wv2, bv2, g2, be2,
                     1, z16, z128)

    a, b = _ab_matmul(h2, Wm1[:HID], Wm1[HID:], bm1)
    z = _make_edgemlp()(a, b, src, dst)
    return _matmul(z, Wm2, bm2, mblk=4000)


# speedup vs baseline: 1.0890x; 1.0890x over previous
import functools

import jax
import jax.numpy as jnp
from jax import lax
from jax.experimental import pallas as pl
from jax.experimental.pallas import tpu as pltpu
from jax.experimental.pallas import tpu_sc as plsc

N_NODES = 10000
N_EDGES = 320000
D_IN = 128
HEADS1 = 3
HID = 256

NC, NS, NW = 2, 16, 32
MBLK = 1000
EB = 80
NPT = N_NODES // NS

_mesh = plsc.VectorSubcoreMesh(core_axis_name="c", subcore_axis_name="s")


def _iota16():
    return jnp.arange(16, dtype=jnp.int32)


def _make_logits(heads):
    epw = N_EDGES // NW
    nb = epw // EB
    scale = 1.0 / (HID ** 0.5)

    @functools.partial(
        pl.kernel, mesh=_mesh, name=f"k1_logits_h{heads}",
        compiler_params=pltpu.CompilerParams(use_tc_tiling_on_sc=False, needs_layout_passes=False),
        out_type=jax.ShapeDtypeStruct((heads * N_EDGES,), jnp.float32),
        scratch_types=[
            pltpu.VMEM((2, EB), jnp.int32),
            pltpu.VMEM((2, EB), jnp.int32),
            pltpu.VMEM((2 * EB, HID), jnp.float32),
            pltpu.VMEM((2 * EB, HID), jnp.float32),
            pltpu.VMEM((EB,), jnp.float32),
            pltpu.VMEM((16, 16), jnp.float32),
            pltpu.SemaphoreType.DMA((2,)),
            pltpu.SemaphoreType.DMA((2,)),
        ])
    def k(q_hbm, k_hbm, src_hbm, dst_hbm, out_hbm,
          dstv2, srcv2, qrows2, krows2, lstage, tmp, isem, gsem):
        wid = lax.axis_index("s") * NC + lax.axis_index("c")
        base = wid * epw

        def issue_ids(b, p):
            e0 = base + b * EB
            pltpu.async_copy(dst_hbm.at[pl.ds(e0, EB)], dstv2.at[p], isem.at[p])
            pltpu.async_copy(src_hbm.at[pl.ds(e0, EB)], srcv2.at[p], isem.at[p])

        def drain_ids(p):
            pltpu.make_async_copy(dst_hbm.at[pl.ds(0, EB)], dstv2.at[p], isem.at[p]).wait()
            pltpu.make_async_copy(src_hbm.at[pl.ds(0, EB)], srcv2.at[p], isem.at[p]).wait()

        def issue_g(pg, pb, h):
            pltpu.async_copy(q_hbm.at[h].at[dstv2.at[pb]],
                             qrows2.at[pl.ds(pg * EB, EB)], gsem.at[pg])
            pltpu.async_copy(k_hbm.at[h].at[srcv2.at[pb]],
                             krows2.at[pl.ds(pg * EB, EB)], gsem.at[pg])

        def drain_g(pg, pb, h):
            pltpu.make_async_copy(q_hbm.at[h].at[dstv2.at[pb]],
                                  qrows2.at[pl.ds(pg * EB, EB)], gsem.at[pg]).wait()
            pltpu.make_async_copy(k_hbm.at[h].at[srcv2.at[pb]],
                                  krows2.at[pl.ds(pg * EB, EB)], gsem.at[pg]).wait()

        def compute(b, h, pg):
            e0 = base + b * EB
            rbase = pg * EB

            def group(g, cc):
                @plsc.parallel_loop(0, 16, unroll=4)
                def edge(jj):
                    j = rbase + g * 16 + jj
                    sl0 = pl.ds(0, 16)
                    pv = qrows2[j, sl0] * krows2[j, sl0]
                    for c16 in range(1, HID // 16):
                        sl = pl.ds(c16 * 16, 16)
                        pv = pv + qrows2[j, sl] * krows2[j, sl]
                    plsc.store_scatter(
                        tmp, [_iota16(), jnp.full((16,), 0, jnp.int32) + jj], pv)
                t8 = [tmp[r, :] + tmp[r + 8, :] for r in range(8)]
                t4 = [t8[r] + t8[r + 4] for r in range(4)]
                t2 = [t4[r] + t4[r + 2] for r in range(2)]
                tot = t2[0] + t2[1]
                lstage[pl.ds(g * 16, 16)] = tot * scale
                return cc

            lax.fori_loop(0, EB // 16, group, 0)
            pltpu.sync_copy(lstage, out_hbm.at[pl.ds(h * N_EDGES + e0, EB)])

        issue_ids(0, 0)
        if nb > 1:
            issue_ids(1, 1)
        drain_ids(0)
        issue_g(0, 0, 0)

        def batch(b, carry):
            pb = lax.rem(b, 2)
            pbn = 1 - pb
            for h in range(heads):
                pg = lax.rem(b + h, 2)
                pgn = 1 - pg
                if h + 1 < heads:
                    issue_g(pgn, pb, h + 1)
                else:
                    @pl.when(b + 1 < nb)
                    def _():
                        drain_ids(pbn)
                        issue_g(pgn, pbn, 0)
                drain_g(pg, pb, h)
                if h == heads - 1:
                    @pl.when(b + 2 < nb)
                    def _():
                        issue_ids(b + 2, pb)
                compute(b, h, pg)
            return carry

        lax.fori_loop(0, nb, batch, 0)

    return k


def _gmax_body(l_ref, m_ref):
    i = pl.program_id(0)
    h = l_ref.shape[0]
    bm = jnp.max(l_ref[...], axis=1, keepdims=True)
    bm = jnp.broadcast_to(bm, (h, 128))
    bm = jnp.concatenate([bm, jnp.zeros((8 - h, 128), jnp.float32)], axis=0)

    @pl.when(i == 0)
    def _():
        m_ref[...] = bm

    @pl.when(i != 0)
    def _():
        m_ref[...] = jnp.maximum(m_ref[...], bm)


def _gmax(logits_flat, heads):
    l2 = logits_flat.reshape(heads, N_EDGES)
    eb = 2560
    return pl.pallas_call(
        _gmax_body,
        grid=(N_EDGES // eb,),
        in_specs=[pl.BlockSpec((heads, eb), lambda i: (0, i))],
        out_specs=pl.BlockSpec((8, 128), lambda i: (0, 0)),
        out_shape=jax.ShapeDtypeStruct((8, 128), jnp.float32),
    )(l2).reshape(1024)


def _make_denom(heads):
    ept = N_EDGES // NW
    nb = ept // EB

    @functools.partial(
        pl.kernel, mesh=_mesh, name=f"k2_denom_h{heads}",
        compiler_params=pltpu.CompilerParams(use_tc_tiling_on_sc=False, needs_layout_passes=False),
        out_type=jax.ShapeDtypeStruct((NC * N_NODES, 16), jnp.float32),
        scratch_types=[
            pltpu.VMEM((EB,), jnp.int32),
            pltpu.VMEM((EB,), jnp.float32),
            pltpu.VMEM((EB, 16), jnp.float32),
            pltpu.VMEM((16,), jnp.float32),
            pltpu.VMEM_SHARED((N_NODES, 16), jnp.float32),
            pltpu.SemaphoreType.DMA,
        ])
    def k(l_hbm, m_hbm, dst_hbm, z_hbm, out_hbm,
          dstv, lbuf, estage, mbuf, dacc, sem):
        c = lax.axis_index("c")
        s = lax.axis_index("s")
        pltpu.sync_copy(z_hbm, dacc.at[pl.ds(s * NPT, NPT)])
        pltpu.sync_copy(z_hbm.at[pl.ds(0, EB)], estage)
        plsc.subcore_barrier()

        mvals = []
        for h in range(heads):
            pltpu.sync_copy(m_hbm.at[pl.ds(h * 128, 16)], mbuf)
            mvals.append(mbuf[...])

        base = (c * NS + s) * ept

        def batch(b, carry):
            e0 = base + b * EB
            pltpu.sync_copy(dst_hbm.at[pl.ds(e0, EB)], dstv)
            for h in range(heads):
                pltpu.sync_copy(l_hbm.at[pl.ds(h * N_EDGES + e0, EB)], lbuf)
                for g in range(EB // 16):
                    rows = _iota16() + (g * 16)
                    ev = jnp.exp(lbuf[pl.ds(g * 16, 16)] - mvals[h])
                    plsc.store_scatter(estage,
                                       [rows, jnp.full((16,), h, jnp.int32)], ev)
            pltpu.sync_copy(estage, dacc.at[dstv], add=True)
            return carry

        lax.fori_loop(0, nb, batch, 0)
        plsc.subcore_barrier()
        pltpu.sync_copy(dacc.at[pl.ds(s * NPT, NPT)],
                        out_hbm.at[pl.ds(c * N_NODES + s * NPT, NPT)])

    return k


def _dencomb_body(d_ref, r_ref):
    d = d_ref[0] + d_ref[1]
    r_ref[...] = 1.0 / (d + 1e-16)


def _dencomb(denoms):
    d3 = denoms.reshape(NC, N_NODES, 16)
    return pl.pallas_call(
        _dencomb_body,
        grid=(N_NODES // MBLK,),
        in_specs=[pl.BlockSpec((NC, MBLK, 16), lambda i: (0, i, 0))],
        out_specs=pl.BlockSpec((MBLK, 16), lambda i: (i, 0)),
        out_shape=jax.ShapeDtypeStruct((N_NODES, 16), jnp.float32),
    )(d3)


def _make_msg(heads):
    chunks = 2 * heads
    chs = chunks // NC
    ept = N_EDGES // NS
    nb = ept // EB

    @functools.partial(
        pl.kernel, mesh=_mesh, name=f"k3_msg_h{heads}",
        compiler_params=pltpu.CompilerParams(use_tc_tiling_on_sc=False, needs_layout_passes=False),
        out_type=jax.ShapeDtypeStruct((chunks * N_NODES, 128), jnp.float32),
        scratch_types=[
            pltpu.VMEM((2, EB), jnp.int32),
            pltpu.VMEM((2, EB), jnp.int32),
            pltpu.VMEM((2, EB), jnp.int32),
            pltpu.VMEM((2 * EB, 128), jnp.float32),
            pltpu.VMEM((2, EB), jnp.float32),
            pltpu.VMEM((2 * EB, 16), jnp.float32),
            pltpu.VMEM((EB,), jnp.float32),
            pltpu.VMEM((16,), jnp.float32),
            pltpu.VMEM_SHARED((N_NODES, 128), jnp.float32),
            pltpu.SemaphoreType.DMA((2,)),
            pltpu.SemaphoreType.DMA((2,)),
        ])
    def k(v_hbm, src_hbm, dst_hbm, l_hbm, m_hbm, rd_hbm, z_hbm, out_hbm,
          srcv2, dstv2, vidx2, vrows2, lbuf, rdrows2, abuf, mbuf, acc,
          isem, gsem):
        c = lax.axis_index("c")
        s = lax.axis_index("s")
        base_e = s * ept

        for t in range(chs):
            ch = c * chs + t
            h = ch // 2
            pltpu.sync_copy(z_hbm, acc.at[pl.ds(s * NPT, NPT)])
            plsc.subcore_barrier()
            pltpu.sync_copy(m_hbm.at[pl.ds(h * 128, 16)], mbuf)
            mh = mbuf[...]
            hcols = jnp.full((16,), 0, jnp.int32) + h

            def issue_ids(b, p):
                e0 = base_e + b * EB
                pltpu.async_copy(src_hbm.at[pl.ds(e0, EB)], srcv2.at[p], isem.at[p])
                pltpu.async_copy(dst_hbm.at[pl.ds(e0, EB)], dstv2.at[p], isem.at[p])
                pltpu.async_copy(l_hbm.at[pl.ds(h * N_EDGES + e0, EB)],
                                 lbuf.at[p], isem.at[p])

            def drain_ids(p):
                pltpu.make_async_copy(src_hbm.at[pl.ds(0, EB)], srcv2.at[p], isem.at[p]).wait()
                pltpu.make_async_copy(dst_hbm.at[pl.ds(0, EB)], dstv2.at[p], isem.at[p]).wait()
                pltpu.make_async_copy(l_hbm.at[pl.ds(0, EB)], lbuf.at[p], isem.at[p]).wait()

            def issue_g(p):
                for g in range(EB // 16):
                    sl = pl.ds(g * 16, 16)
                    vidx2[p, sl] = srcv2[p, sl] + ch * N_NODES
                pltpu.async_copy(v_hbm.at[vidx2.at[p]],
                                 vrows2.at[pl.ds(p * EB, EB)], gsem.at[p])
                pltpu.async_copy(rd_hbm.at[dstv2.at[p]],
                                 rdrows2.at[pl.ds(p * EB, EB)], gsem.at[p])

            def drain_g(p):
                pltpu.make_async_copy(v_hbm.at[vidx2.at[p]],
                                      vrows2.at[pl.ds(p * EB, EB)], gsem.at[p]).wait()
                pltpu.make_async_copy(rd_hbm.at[dstv2.at[p]],
                                      rdrows2.at[pl.ds(p * EB, EB)], gsem.at[p]).wait()

            def compute(b, p):
                rbase = p * EB
                rrows0 = _iota16() + rbase
                for g in range(EB // 16):
                    rows = rrows0 + (g * 16)
                    ev = jnp.exp(lbuf[p, pl.ds(g * 16, 16)] - mh)
                    rd = plsc.load_gather(rdrows2, [rows, hcols])
                    abuf[pl.ds(g * 16, 16)] = ev * rd

                @plsc.parallel_loop(0, EB, unroll=8)
                def edge(j):
                    asp = plsc.load_gather(abuf, [jnp.full((16,), 0, jnp.int32) + j])
                    jr = rbase + j
                    for c8 in range(8):
                        sl = pl.ds(c8 * 16, 16)
                        vrows2[jr, sl] = vrows2[jr, sl] * asp
                pltpu.sync_copy(vrows2.at[pl.ds(p * EB, EB)],
                                acc.at[dstv2.at[p]], add=True)

            issue_ids(0, 0)
            issue_ids(1, 1)
            drain_ids(0)
            issue_g(0)

            def batch(b, carry):
                p = lax.rem(b, 2)
                pn = 1 - p

                @pl.when(b + 1 < nb)
                def _():
                    drain_ids(pn)
                    issue_g(pn)

                drain_g(p)
                compute(b, p)

                @pl.when(b + 2 < nb)
                def _():
                    issue_ids(b + 2, p)

                return carry

            lax.fori_loop(0, nb, batch, 0)
            plsc.subcore_barrier()
            pltpu.sync_copy(acc.at[pl.ds(s * NPT, NPT)],
                            out_hbm.at[pl.ds(ch * N_NODES + s * NPT, NPT)])

    return k


def _make_edgemlp():
    epw = N_EDGES // NW
    nb = epw // EB

    @functools.partial(
        pl.kernel, mesh=_mesh, name="k4_edgemlp",
        compiler_params=pltpu.CompilerParams(use_tc_tiling_on_sc=False, needs_layout_passes=False),
        out_type=jax.ShapeDtypeStruct((N_EDGES, HID), jnp.float32),
        scratch_types=[
            pltpu.VMEM((2, EB), jnp.int32),
            pltpu.VMEM((2, EB), jnp.int32),
            pltpu.VMEM((2 * EB, HID), jnp.float32),
            pltpu.VMEM((2 * EB, HID), jnp.float32),
            pltpu.SemaphoreType.DMA((2,)),
            pltpu.SemaphoreType.DMA((2,)),
        ])
    def k(a_hbm, b_hbm, src_hbm, dst_hbm, z_hbm,
          srcv2, dstv2, arows2, brows2, isem, gsem):
        wid = lax.axis_index("s") * NC + lax.axis_index("c")
        base = wid * epw

        def issue_ids(b, p):
            e0 = base + b * EB
            pltpu.async_copy(src_hbm.at[pl.ds(e0, EB)], srcv2.at[p], isem.at[p])
            pltpu.async_copy(dst_hbm.at[pl.ds(e0, EB)], dstv2.at[p], isem.at[p])

        def drain_ids(p):
            pltpu.make_async_copy(src_hbm.at[pl.ds(0, EB)], srcv2.at[p], isem.at[p]).wait()
            pltpu.make_async_copy(dst_hbm.at[pl.ds(0, EB)], dstv2.at[p], isem.at[p]).wait()

        def issue_g(p):
            pltpu.async_copy(a_hbm.at[srcv2.at[p]],
                             arows2.at[pl.ds(p * EB, EB)], gsem.at[p])
            pltpu.async_copy(b_hbm.at[dstv2.at[p]],
                             brows2.at[pl.ds(p * EB, EB)], gsem.at[p])

        def drain_g(p):
            pltpu.make_async_copy(a_hbm.at[srcv2.at[p]],
                                  arows2.at[pl.ds(p * EB, EB)], gsem.at[p]).wait()
            pltpu.make_async_copy(b_hbm.at[dstv2.at[p]],
                                  brows2.at[pl.ds(p * EB, EB)], gsem.at[p]).wait()

        def compute(b, p):
            e0 = base + b * EB
            rbase = p * EB

            @plsc.parallel_loop(0, EB, unroll=8)
            def edge(j):
                jr = rbase + j
                for c16 in range(HID // 16):
                    sl = pl.ds(c16 * 16, 16)
                    arows2[jr, sl] = jnp.maximum(
                        arows2[jr, sl] + brows2[jr, sl], 0.0)
            pltpu.sync_copy(arows2.at[pl.ds(p * EB, EB)], z_hbm.at[pl.ds(e0, EB)])

        issue_ids(0, 0)
        issue_ids(1, 1)
        drain_ids(0)
        issue_g(0)

        def batch(b, carry):
            p = lax.rem(b, 2)
            pn = 1 - p

            @pl.when(b + 1 < nb)
            def _():
                drain_ids(pn)
                issue_g(pn)

            drain_g(p)
            compute(b, p)

            @pl.when(b + 2 < nb)
            def _():
                issue_ids(b + 2, p)

            return carry

        lax.fori_loop(0, nb, batch, 0)

    return k


def _proj_body(x_ref, wq_ref, bq_ref, wk_ref, bk_ref, wv_ref, bv_ref,
               q_ref, k_ref, v_ref, *, heads):
    x = x_ref[...]
    for h in range(heads):
        sl = slice(h * HID, (h + 1) * HID)
        q_ref[h] = jnp.dot(x, wq_ref[:, sl], preferred_element_type=jnp.float32) + bq_ref[0, sl]
        k_ref[h] = jnp.dot(x, wk_ref[:, sl], preferred_element_type=jnp.float32) + bk_ref[0, sl]
    for ch in range(2 * heads):
        sl = slice(ch * 128, (ch + 1) * 128)
        v_ref[ch] = jnp.dot(x, wv_ref[:, sl], preferred_element_type=jnp.float32) + bv_ref[0, sl]


def _proj(x, wq, bq, wk, bk, wv, bv, heads):
    n, d = x.shape
    dout = heads * HID
    grid = (n // MBLK,)
    bspec_w = pl.BlockSpec((d, dout), lambda i: (0, 0))
    bspec_b = pl.BlockSpec((1, dout), lambda i: (0, 0))
    q, k, v = pl.pallas_call(
        functools.partial(_proj_body, heads=heads),
        grid=grid,
        in_specs=[pl.BlockSpec((MBLK, d), lambda i: (i, 0)),
                  bspec_w, bspec_b, bspec_w, bspec_b, bspec_w, bspec_b],
        out_specs=[pl.BlockSpec((heads, MBLK, HID), lambda i: (0, i, 0)),
                   pl.BlockSpec((heads, MBLK, HID), lambda i: (0, i, 0)),
                   pl.BlockSpec((2 * heads, MBLK, 128), lambda i: (0, i, 0))],
        out_shape=[jax.ShapeDtypeStruct((heads, n, HID), jnp.float32),
                   jax.ShapeDtypeStruct((heads, n, HID), jnp.float32),
                   jax.ShapeDtypeStruct((2 * heads, n, 128), jnp.float32)],
    )(x, wq, bq.reshape(1, dout), wk, bk.reshape(1, dout), wv, bv.reshape(1, dout))
    return q, k, v.reshape(2 * heads * n, 128)


def _bnstats_body(o_ref, s_ref, ss_ref):
    o = o_ref[...]
    s_ref[...] = jnp.sum(o, axis=1)[None]
    ss_ref[...] = jnp.sum(o * o, axis=1)[None]


def _bnapply_body(o_ref, s_ref, ss_ref, g_ref, be_ref, out_ref, *, n_total):
    ch = o_ref.shape[0]
    mean = jnp.sum(s_ref[...], axis=0) / n_total
    ex2 = jnp.sum(ss_ref[...], axis=0) / n_total
    var = ex2 - mean * mean
    scale = lax.rsqrt(var + 1e-5) * g_ref[...]
    shift = be_ref[...] - mean * scale
    o = o_ref[...] * scale[:, None, :] + shift[:, None, :]
    o = jnp.maximum(o, 0.0)
    out_ref[...] = o.transpose(1, 0, 2).reshape(o_ref.shape[1], ch * 128)


def _bn_relu(opre_flat, g, be, ch):
    opre = opre_flat.reshape(ch, N_NODES, 128)
    n = N_NODES
    grid = (n // MBLK,)
    nblk = n // MBLK
    s, ss = pl.pallas_call(
        _bnstats_body,
        grid=grid,
        in_specs=[pl.BlockSpec((ch, MBLK, 128), lambda i: (0, i, 0))],
        out_specs=[pl.BlockSpec((1, ch, 128), lambda i: (i, 0, 0)),
                   pl.BlockSpec((1, ch, 128), lambda i: (i, 0, 0))],
        out_shape=[jax.ShapeDtypeStruct((nblk, ch, 128), jnp.float32),
                   jax.ShapeDtypeStruct((nblk, ch, 128), jnp.float32)],
    )(opre)
    return pl.pallas_call(
        functools.partial(_bnapply_body, n_total=float(n)),
        grid=grid,
        in_specs=[pl.BlockSpec((ch, MBLK, 128), lambda i: (0, i, 0)),
                  pl.BlockSpec((nblk, ch, 128), lambda i: (0, 0, 0)),
                  pl.BlockSpec((nblk, ch, 128), lambda i: (0, 0, 0)),
                  pl.BlockSpec((ch, 128), lambda i: (0, 0)),
                  pl.BlockSpec((ch, 128), lambda i: (0, 0))],
        out_specs=pl.BlockSpec((MBLK, ch * 128), lambda i: (i, 0)),
        out_shape=jax.ShapeDtypeStruct((n, ch * 128), jnp.float32),
    )(opre, s, ss, g.reshape(ch, 128), be.reshape(ch, 128))


def _mm2_body(x_ref, wa_ref, wb_ref, bb_ref, a_ref, b_ref):
    x = x_ref[...]
    a_ref[...] = jnp.dot(x, wa_ref[...], preferred_element_type=jnp.float32)
    b_ref[...] = jnp.dot(x, wb_ref[...], preferred_element_type=jnp.float32) + bb_ref[...]


def _ab_matmul(x, wa, wb, bb):
    n, d = x.shape
    dout = wa.shape[1]
    sd = jax.ShapeDtypeStruct((n, dout), jnp.float32)
    return pl.pallas_call(
        _mm2_body,
        grid=(n // MBLK,),
        in_specs=[pl.BlockSpec((MBLK, d), lambda i: (i, 0)),
                  pl.BlockSpec((d, dout), lambda i: (0, 0)),
                  pl.BlockSpec((d, dout), lambda i: (0, 0)),
                  pl.BlockSpec((1, dout), lambda i: (0, 0))],
        out_specs=[pl.BlockSpec((MBLK, dout), lambda i: (i, 0)),
                   pl.BlockSpec((MBLK, dout), lambda i: (i, 0))],
        out_shape=[sd, sd],
    )(x, wa, wb, bb.reshape(1, dout))


def _mm_body(x_ref, w_ref, b_ref, o_ref):
    o_ref[...] = jnp.dot(x_ref[...], w_ref[...],
                         preferred_element_type=jnp.float32) + b_ref[...]


def _matmul(x, w, b, mblk=MBLK):
    n, d = x.shape
    dout = w.shape[1]
    return pl.pallas_call(
        _mm_body,
        grid=(n // mblk,),
        in_specs=[pl.BlockSpec((mblk, d), lambda i: (i, 0)),
                  pl.BlockSpec((d, dout), lambda i: (0, 0)),
                  pl.BlockSpec((1, dout), lambda i: (0, 0))],
        out_specs=pl.BlockSpec((mblk, dout), lambda i: (i, 0)),
        out_shape=jax.ShapeDtypeStruct((n, dout), jnp.float32),
    )(x, w, b.reshape(1, dout))


def _tgcn_layer(x, src, dst, wq, bq, wk, bk, wv, bv, g, be, heads,
                z16, z128):
    q2d, k2d, v2d = _proj(x, wq, bq, wk, bk, wv, bv, heads)
    logits = _make_logits(heads)(q2d, k2d, src, dst)
    m = _gmax(logits, heads)
    denoms = _make_denom(heads)(logits, m, dst, z16)
    rden = _dencomb(denoms)
    opre = _make_msg(heads)(v2d, src, dst, logits, m, rden, z128)
    return _bn_relu(opre, g, be, 2 * heads)


def kernel(x, edge_index, Wq1, bq1, Wk1, bk1, Wv1, bv1, g1, be1,
           Wq2, bq2, Wk2, bk2, Wv2, bv2, g2, be2, Wm1, bm1, Wm2, bm2):
    src = edge_index[0]
    dst = edge_index[1]
    z16 = jnp.zeros((NPT, 16), jnp.float32)
    z128 = jnp.zeros((NPT, 128), jnp.float32)

    h1 = _tgcn_layer(x, src, dst, Wq1, bq1, Wk1, bk1, Wv1, bv1, g1, be1,
                     HEADS1, z16, z128)
    h2 = _tgcn_layer(h1, src, dst, Wq2, bq2, Wk2, bk2, Wv2, bv2, g2, be2,
                     1, z16, z128)

    a, b = _ab_matmul(h2, Wm1[:HID], Wm1[HID:], bm1)
    z = _make_edgemlp()(a, b, src, dst)
    return _matmul(z, Wm2, bm2, mblk=4000)


# speedup vs baseline: 1.1647x; 1.0695x over previous
import functools

import jax
import jax.numpy as jnp
from jax import lax
from jax.experimental import pallas as pl
from jax.experimental.pallas import tpu as pltpu
from jax.experimental.pallas import tpu_sc as plsc

N_NODES = 10000
N_EDGES = 320000
D_IN = 128
HEADS1 = 3
HID = 256

NC, NS, NW = 2, 16, 32
MBLK = 1000
EB = 80
NPT = N_NODES // NS

_mesh = plsc.VectorSubcoreMesh(core_axis_name="c", subcore_axis_name="s")


def _iota16():
    return jnp.arange(16, dtype=jnp.int32)


def _make_logits(heads):
    epw = N_EDGES // NW
    nb = epw // EB
    scale = 1.0 / (HID ** 0.5)

    @functools.partial(
        pl.kernel, mesh=_mesh, name=f"k1_logits_h{heads}",
        compiler_params=pltpu.CompilerParams(use_tc_tiling_on_sc=False, needs_layout_passes=False),
        out_type=jax.ShapeDtypeStruct((heads * N_EDGES,), jnp.float32),
        scratch_types=[
            pltpu.VMEM((2, EB), jnp.int32),
            pltpu.VMEM((2, EB), jnp.int32),
            pltpu.VMEM((2 * EB, HID), jnp.float32),
            pltpu.VMEM((2 * EB, HID), jnp.float32),
            pltpu.VMEM((EB,), jnp.float32),
            pltpu.VMEM((16, 16), jnp.float32),
            pltpu.SemaphoreType.DMA((2,)),
            pltpu.SemaphoreType.DMA((2,)),
        ])
    def k(q_hbm, k_hbm, src_hbm, dst_hbm, out_hbm,
          dstv2, srcv2, qrows2, krows2, lstage, tmp, isem, gsem):
        wid = lax.axis_index("s") * NC + lax.axis_index("c")
        base = wid * epw

        def issue_ids(b, p):
            e0 = base + b * EB
            pltpu.async_copy(dst_hbm.at[pl.ds(e0, EB)], dstv2.at[p], isem.at[p])
            pltpu.async_copy(src_hbm.at[pl.ds(e0, EB)], srcv2.at[p], isem.at[p])

        def drain_ids(p):
            pltpu.make_async_copy(dst_hbm.at[pl.ds(0, EB)], dstv2.at[p], isem.at[p]).wait()
            pltpu.make_async_copy(src_hbm.at[pl.ds(0, EB)], srcv2.at[p], isem.at[p]).wait()

        def issue_g(pg, pb, h):
            pltpu.async_copy(q_hbm.at[h].at[dstv2.at[pb]],
                             qrows2.at[pl.ds(pg * EB, EB)], gsem.at[pg])
            pltpu.async_copy(k_hbm.at[h].at[srcv2.at[pb]],
                             krows2.at[pl.ds(pg * EB, EB)], gsem.at[pg])

        def drain_g(pg, pb, h):
            pltpu.make_async_copy(q_hbm.at[h].at[dstv2.at[pb]],
                                  qrows2.at[pl.ds(pg * EB, EB)], gsem.at[pg]).wait()
            pltpu.make_async_copy(k_hbm.at[h].at[srcv2.at[pb]],
                                  krows2.at[pl.ds(pg * EB, EB)], gsem.at[pg]).wait()

        def compute(b, h, pg):
            e0 = base + b * EB
            rbase = pg * EB

            def group(g, cc):
                @plsc.parallel_loop(0, 16, unroll=4)
                def edge(jj):
                    j = rbase + g * 16 + jj
                    sl0 = pl.ds(0, 16)
                    pv = qrows2[j, sl0] * krows2[j, sl0]
                    for c16 in range(1, HID // 16):
                        sl = pl.ds(c16 * 16, 16)
                        pv = pv + qrows2[j, sl] * krows2[j, sl]
                    plsc.store_scatter(
                        tmp, [_iota16(), jnp.full((16,), 0, jnp.int32) + jj], pv)
                t8 = [tmp[r, :] + tmp[r + 8, :] for r in range(8)]
                t4 = [t8[r] + t8[r + 4] for r in range(4)]
                t2 = [t4[r] + t4[r + 2] for r in range(2)]
                tot = t2[0] + t2[1]
                lstage[pl.ds(g * 16, 16)] = tot * scale
                return cc

            lax.fori_loop(0, EB // 16, group, 0)
            pltpu.sync_copy(lstage, out_hbm.at[pl.ds(h * N_EDGES + e0, EB)])

        issue_ids(0, 0)
        if nb > 1:
            issue_ids(1, 1)
        drain_ids(0)
        issue_g(0, 0, 0)

        def batch(b, carry):
            pb = lax.rem(b, 2)
            pbn = 1 - pb
            for h in range(heads):
                pg = lax.rem(b + h, 2)
                pgn = 1 - pg
                if h + 1 < heads:
                    issue_g(pgn, pb, h + 1)
                else:
                    @pl.when(b + 1 < nb)
                    def _():
                        drain_ids(pbn)
                        issue_g(pgn, pbn, 0)
                drain_g(pg, pb, h)
                if h == heads - 1:
                    @pl.when(b + 2 < nb)
                    def _():
                        issue_ids(b + 2, pb)
                compute(b, h, pg)
            return carry

        lax.fori_loop(0, nb, batch, 0)

    return k


def _gmax_body(l_ref, m_ref):
    i = pl.program_id(0)
    h = l_ref.shape[0]
    bm = jnp.max(l_ref[...], axis=1, keepdims=True)
    bm = jnp.broadcast_to(bm, (h, 128))
    bm = jnp.concatenate([bm, jnp.zeros((8 - h, 128), jnp.float32)], axis=0)

    @pl.when(i == 0)
    def _():
        m_ref[...] = bm

    @pl.when(i != 0)
    def _():
        m_ref[...] = jnp.maximum(m_ref[...], bm)


def _gmax(logits_flat, heads):
    l2 = logits_flat.reshape(heads, N_EDGES)
    eb = 2560
    return pl.pallas_call(
        _gmax_body,
        grid=(N_EDGES // eb,),
        in_specs=[pl.BlockSpec((heads, eb), lambda i: (0, i))],
        out_specs=pl.BlockSpec((8, 128), lambda i: (0, 0)),
        out_shape=jax.ShapeDtypeStruct((8, 128), jnp.float32),
    )(l2).reshape(1024)


def _make_denom(heads):
    ept = N_EDGES // NW
    nb = ept // EB

    @functools.partial(
        pl.kernel, mesh=_mesh, name=f"k2_denom_h{heads}",
        compiler_params=pltpu.CompilerParams(use_tc_tiling_on_sc=False, needs_layout_passes=False),
        out_type=jax.ShapeDtypeStruct((NC * N_NODES, 16), jnp.float32),
        scratch_types=[
            pltpu.VMEM((2, EB), jnp.int32),
            pltpu.VMEM((2 * heads, EB), jnp.float32),
            pltpu.VMEM((EB, 16), jnp.float32),
            pltpu.VMEM((16,), jnp.float32),
            pltpu.VMEM_SHARED((N_NODES, 16), jnp.float32),
            pltpu.SemaphoreType.DMA((2,)),
        ])
    def k(l_hbm, m_hbm, dst_hbm, z_hbm, out_hbm,
          dstv2, lb2, estage, mbuf, dacc, isem):
        c = lax.axis_index("c")
        s = lax.axis_index("s")
        pltpu.sync_copy(z_hbm, dacc.at[pl.ds(s * NPT, NPT)])
        pltpu.sync_copy(z_hbm.at[pl.ds(0, EB)], estage)
        plsc.subcore_barrier()

        mvals = []
        for h in range(heads):
            pltpu.sync_copy(m_hbm.at[pl.ds(h * 128, 16)], mbuf)
            mvals.append(mbuf[...])

        base = (c * NS + s) * ept

        def issue(b, p):
            e0 = base + b * EB
            pltpu.async_copy(dst_hbm.at[pl.ds(e0, EB)], dstv2.at[p], isem.at[p])
            for h in range(heads):
                pltpu.async_copy(l_hbm.at[pl.ds(h * N_EDGES + e0, EB)],
                                 lb2.at[p * heads + h], isem.at[p])

        def drain(p):
            pltpu.make_async_copy(dst_hbm.at[pl.ds(0, EB)], dstv2.at[p], isem.at[p]).wait()
            for h in range(heads):
                pltpu.make_async_copy(l_hbm.at[pl.ds(0, EB)],
                                      lb2.at[p * heads + h], isem.at[p]).wait()

        issue(0, 0)
        if nb > 1:
            issue(1, 1)

        def batch(b, carry):
            p = lax.rem(b, 2)
            drain(p)
            for h in range(heads):
                for g in range(EB // 16):
                    rows = _iota16() + (g * 16)
                    ev = jnp.exp(lb2[p * heads + h, pl.ds(g * 16, 16)] - mvals[h])
                    plsc.store_scatter(estage,
                                       [rows, jnp.full((16,), h, jnp.int32)], ev)
            pltpu.sync_copy(estage, dacc.at[dstv2.at[p]], add=True)

            @pl.when(b + 2 < nb)
            def _():
                issue(b + 2, p)

            return carry

        lax.fori_loop(0, nb, batch, 0)
        plsc.subcore_barrier()
        pltpu.sync_copy(dacc.at[pl.ds(s * NPT, NPT)],
                        out_hbm.at[pl.ds(c * N_NODES + s * NPT, NPT)])

    return k


def _dencomb_body(d_ref, r_ref):
    d = d_ref[0] + d_ref[1]
    r_ref[...] = 1.0 / (d + 1e-16)


def _dencomb(denoms):
    d3 = denoms.reshape(NC, N_NODES, 16)
    return pl.pallas_call(
        _dencomb_body,
        grid=(N_NODES // MBLK,),
        in_specs=[pl.BlockSpec((NC, MBLK, 16), lambda i: (0, i, 0))],
        out_specs=pl.BlockSpec((MBLK, 16), lambda i: (i, 0)),
        out_shape=jax.ShapeDtypeStruct((N_NODES, 16), jnp.float32),
    )(d3)


def _make_msg(heads):
    chunks = 2 * heads
    chs = chunks // NC
    ept = N_EDGES // NS
    nb = ept // EB

    @functools.partial(
        pl.kernel, mesh=_mesh, name=f"k3_msg_h{heads}",
        compiler_params=pltpu.CompilerParams(use_tc_tiling_on_sc=False, needs_layout_passes=False),
        out_type=jax.ShapeDtypeStruct((chunks * N_NODES, 128), jnp.float32),
        scratch_types=[
            pltpu.VMEM((2, EB), jnp.int32),
            pltpu.VMEM((2, EB), jnp.int32),
            pltpu.VMEM((2, EB), jnp.int32),
            pltpu.VMEM((2 * EB, 128), jnp.float32),
            pltpu.VMEM((2, EB), jnp.float32),
            pltpu.VMEM((2 * EB, 16), jnp.float32),
            pltpu.VMEM((EB,), jnp.float32),
            pltpu.VMEM((16,), jnp.float32),
            pltpu.VMEM_SHARED((N_NODES, 128), jnp.float32),
            pltpu.SemaphoreType.DMA((2,)),
            pltpu.SemaphoreType.DMA((2,)),
        ])
    def k(v_hbm, src_hbm, dst_hbm, l_hbm, m_hbm, rd_hbm, z_hbm, out_hbm,
          srcv2, dstv2, vidx2, vrows2, lbuf, rdrows2, abuf, mbuf, acc,
          isem, gsem):
        c = lax.axis_index("c")
        s = lax.axis_index("s")
        base_e = s * ept

        for t in range(chs):
            ch = c * chs + t
            h = ch // 2
            pltpu.sync_copy(z_hbm, acc.at[pl.ds(s * NPT, NPT)])
            plsc.subcore_barrier()
            pltpu.sync_copy(m_hbm.at[pl.ds(h * 128, 16)], mbuf)
            mh = mbuf[...]
            hcols = jnp.full((16,), 0, jnp.int32) + h

            def issue_ids(b, p):
                e0 = base_e + b * EB
                pltpu.async_copy(src_hbm.at[pl.ds(e0, EB)], srcv2.at[p], isem.at[p])
                pltpu.async_copy(dst_hbm.at[pl.ds(e0, EB)], dstv2.at[p], isem.at[p])
                pltpu.async_copy(l_hbm.at[pl.ds(h * N_EDGES + e0, EB)],
                                 lbuf.at[p], isem.at[p])

            def drain_ids(p):
                pltpu.make_async_copy(src_hbm.at[pl.ds(0, EB)], srcv2.at[p], isem.at[p]).wait()
                pltpu.make_async_copy(dst_hbm.at[pl.ds(0, EB)], dstv2.at[p], isem.at[p]).wait()
                pltpu.make_async_copy(l_hbm.at[pl.ds(0, EB)], lbuf.at[p], isem.at[p]).wait()

            def issue_g(p):
                for g in range(EB // 16):
                    sl = pl.ds(g * 16, 16)
                    vidx2[p, sl] = srcv2[p, sl] + ch * N_NODES
                pltpu.async_copy(v_hbm.at[vidx2.at[p]],
                                 vrows2.at[pl.ds(p * EB, EB)], gsem.at[p])
                pltpu.async_copy(rd_hbm.at[dstv2.at[p]],
                                 rdrows2.at[pl.ds(p * EB, EB)], gsem.at[p])

            def drain_g(p):
                pltpu.make_async_copy(v_hbm.at[vidx2.at[p]],
                                      vrows2.at[pl.ds(p * EB, EB)], gsem.at[p]).wait()
                pltpu.make_async_copy(rd_hbm.at[dstv2.at[p]],
                                      rdrows2.at[pl.ds(p * EB, EB)], gsem.at[p]).wait()

            def compute(b, p):
                rbase = p * EB
                rrows0 = _iota16() + rbase
                for g in range(EB // 16):
                    rows = rrows0 + (g * 16)
                    ev = jnp.exp(lbuf[p, pl.ds(g * 16, 16)] - mh)
                    rd = plsc.load_gather(rdrows2, [rows, hcols])
                    abuf[pl.ds(g * 16, 16)] = ev * rd

                @plsc.parallel_loop(0, EB, unroll=8)
                def edge(j):
                    asp = plsc.load_gather(abuf, [jnp.full((16,), 0, jnp.int32) + j])
                    jr = rbase + j
                    for c8 in range(8):
                        sl = pl.ds(c8 * 16, 16)
                        vrows2[jr, sl] = vrows2[jr, sl] * asp
                pltpu.sync_copy(vrows2.at[pl.ds(p * EB, EB)],
                                acc.at[dstv2.at[p]], add=True)

            issue_ids(0, 0)
            issue_ids(1, 1)
            drain_ids(0)
            issue_g(0)

            def batch(b, carry):
                p = lax.rem(b, 2)
                pn = 1 - p

                @pl.when(b + 1 < nb)
                def _():
                    drain_ids(pn)
                    issue_g(pn)

                drain_g(p)
                compute(b, p)

                @pl.when(b + 2 < nb)
                def _():
                    issue_ids(b + 2, p)

                return carry

            lax.fori_loop(0, nb, batch, 0)
            plsc.subcore_barrier()
            pltpu.sync_copy(acc.at[pl.ds(s * NPT, NPT)],
                            out_hbm.at[pl.ds(ch * N_NODES + s * NPT, NPT)])

    return k


def _make_edgemlp():
    epw = N_EDGES // NW
    nb = epw // EB

    @functools.partial(
        pl.kernel, mesh=_mesh, name="k4_edgemlp",
        compiler_params=pltpu.CompilerParams(use_tc_tiling_on_sc=False, needs_layout_passes=False),
        out_type=jax.ShapeDtypeStruct((N_EDGES, HID), jnp.float32),
        scratch_types=[
            pltpu.VMEM((2, EB), jnp.int32),
            pltpu.VMEM((2, EB), jnp.int32),
            pltpu.VMEM((2 * EB, HID), jnp.float32),
            pltpu.VMEM((2 * EB, HID), jnp.float32),
            pltpu.SemaphoreType.DMA((2,)),
            pltpu.SemaphoreType.DMA((2,)),
        ])
    def k(a_hbm, b_hbm, src_hbm, dst_hbm, z_hbm,
          srcv2, dstv2, arows2, brows2, isem, gsem):
        wid = lax.axis_index("s") * NC + lax.axis_index("c")
        base = wid * epw

        def issue_ids(b, p):
            e0 = base + b * EB
            pltpu.async_copy(src_hbm.at[pl.ds(e0, EB)], srcv2.at[p], isem.at[p])
            pltpu.async_copy(dst_hbm.at[pl.ds(e0, EB)], dstv2.at[p], isem.at[p])

        def drain_ids(p):
            pltpu.make_async_copy(src_hbm.at[pl.ds(0, EB)], srcv2.at[p], isem.at[p]).wait()
            pltpu.make_async_copy(dst_hbm.at[pl.ds(0, EB)], dstv2.at[p], isem.at[p]).wait()

        def issue_g(p):
            pltpu.async_copy(a_hbm.at[srcv2.at[p]],
                             arows2.at[pl.ds(p * EB, EB)], gsem.at[p])
            pltpu.async_copy(b_hbm.at[dstv2.at[p]],
                             brows2.at[pl.ds(p * EB, EB)], gsem.at[p])

        def drain_g(p):
            pltpu.make_async_copy(a_hbm.at[srcv2.at[p]],
                                  arows2.at[pl.ds(p * EB, EB)], gsem.at[p]).wait()
            pltpu.make_async_copy(b_hbm.at[dstv2.at[p]],
                                  brows2.at[pl.ds(p * EB, EB)], gsem.at[p]).wait()

        def compute(b, p):
            e0 = base + b * EB
            rbase = p * EB

            @plsc.parallel_loop(0, EB, unroll=8)
            def edge(j):
                jr = rbase + j
                for c16 in range(HID // 16):
                    sl = pl.ds(c16 * 16, 16)
                    arows2[jr, sl] = jnp.maximum(
                        arows2[jr, sl] + brows2[jr, sl], 0.0)
            pltpu.sync_copy(arows2.at[pl.ds(p * EB, EB)], z_hbm.at[pl.ds(e0, EB)])

        issue_ids(0, 0)
        issue_ids(1, 1)
        drain_ids(0)
        issue_g(0)

        def batch(b, carry):
            p = lax.rem(b, 2)
            pn = 1 - p

            @pl.when(b + 1 < nb)
            def _():
                drain_ids(pn)
                issue_g(pn)

            drain_g(p)
            compute(b, p)

            @pl.when(b + 2 < nb)
            def _():
                issue_ids(b + 2, p)

            return carry

        lax.fori_loop(0, nb, batch, 0)

    return k


def _proj_body(x_ref, wq_ref, bq_ref, wk_ref, bk_ref, wv_ref, bv_ref,
               q_ref, k_ref, v_ref, *, heads):
    x = x_ref[...]
    for h in range(heads):
        sl = slice(h * HID, (h + 1) * HID)
        q_ref[h] = jnp.dot(x, wq_ref[:, sl], preferred_element_type=jnp.float32) + bq_ref[0, sl]
        k_ref[h] = jnp.dot(x, wk_ref[:, sl], preferred_element_type=jnp.float32) + bk_ref[0, sl]
    for ch in range(2 * heads):
        sl = slice(ch * 128, (ch + 1) * 128)
        v_ref[ch] = jnp.dot(x, wv_ref[:, sl], preferred_element_type=jnp.float32) + bv_ref[0, sl]


def _proj(x, wq, bq, wk, bk, wv, bv, heads):
    n, d = x.shape
    dout = heads * HID
    grid = (n // MBLK,)
    bspec_w = pl.BlockSpec((d, dout), lambda i: (0, 0))
    bspec_b = pl.BlockSpec((1, dout), lambda i: (0, 0))
    q, k, v = pl.pallas_call(
        functools.partial(_proj_body, heads=heads),
        grid=grid,
        in_specs=[pl.BlockSpec((MBLK, d), lambda i: (i, 0)),
                  bspec_w, bspec_b, bspec_w, bspec_b, bspec_w, bspec_b],
        out_specs=[pl.BlockSpec((heads, MBLK, HID), lambda i: (0, i, 0)),
                   pl.BlockSpec((heads, MBLK, HID), lambda i: (0, i, 0)),
                   pl.BlockSpec((2 * heads, MBLK, 128), lambda i: (0, i, 0))],
        out_shape=[jax.ShapeDtypeStruct((heads, n, HID), jnp.float32),
                   jax.ShapeDtypeStruct((heads, n, HID), jnp.float32),
                   jax.ShapeDtypeStruct((2 * heads, n, 128), jnp.float32)],
    )(x, wq, bq.reshape(1, dout), wk, bk.reshape(1, dout), wv, bv.reshape(1, dout))
    return q, k, v.reshape(2 * heads * n, 128)


def _bnstats_body(o_ref, s_ref, ss_ref):
    o = o_ref[...]
    s_ref[...] = jnp.sum(o, axis=1)[None]
    ss_ref[...] = jnp.sum(o * o, axis=1)[None]


def _bnapply_body(o_ref, s_ref, ss_ref, g_ref, be_ref, out_ref, *, n_total):
    ch = o_ref.shape[0]
    mean = jnp.sum(s_ref[...], axis=0) / n_total
    ex2 = jnp.sum(ss_ref[...], axis=0) / n_total
    var = ex2 - mean * mean
    scale = lax.rsqrt(var + 1e-5) * g_ref[...]
    shift = be_ref[...] - mean * scale
    o = o_ref[...] * scale[:, None, :] + shift[:, None, :]
    o = jnp.maximum(o, 0.0)
    out_ref[...] = o.transpose(1, 0, 2).reshape(o_ref.shape[1], ch * 128)


def _bn_relu(opre_flat, g, be, ch):
    opre = opre_flat.reshape(ch, N_NODES, 128)
    n = N_NODES
    grid = (n // MBLK,)
    nblk = n // MBLK
    s, ss = pl.pallas_call(
        _bnstats_body,
        grid=grid,
        in_specs=[pl.BlockSpec((ch, MBLK, 128), lambda i: (0, i, 0))],
        out_specs=[pl.BlockSpec((1, ch, 128), lambda i: (i, 0, 0)),
                   pl.BlockSpec((1, ch, 128), lambda i: (i, 0, 0))],
        out_shape=[jax.ShapeDtypeStruct((nblk, ch, 128), jnp.float32),
                   jax.ShapeDtypeStruct((nblk, ch, 128), jnp.float32)],
    )(opre)
    return pl.pallas_call(
        functools.partial(_bnapply_body, n_total=float(n)),
        grid=grid,
        in_specs=[pl.BlockSpec((ch, MBLK, 128), lambda i: (0, i, 0)),
                  pl.BlockSpec((nblk, ch, 128), lambda i: (0, 0, 0)),
                  pl.BlockSpec((nblk, ch, 128), lambda i: (0, 0, 0)),
                  pl.BlockSpec((ch, 128), lambda i: (0, 0)),
                  pl.BlockSpec((ch, 128), lambda i: (0, 0))],
        out_specs=pl.BlockSpec((MBLK, ch * 128), lambda i: (i, 0)),
        out_shape=jax.ShapeDtypeStruct((n, ch * 128), jnp.float32),
    )(opre, s, ss, g.reshape(ch, 128), be.reshape(ch, 128))


def _mm2_body(x_ref, wa_ref, wb_ref, bb_ref, a_ref, b_ref):
    x = x_ref[...]
    a_ref[...] = jnp.dot(x, wa_ref[...], preferred_element_type=jnp.float32)
    b_ref[...] = jnp.dot(x, wb_ref[...], preferred_element_type=jnp.float32) + bb_ref[...]


def _ab_matmul(x, wa, wb, bb):
    n, d = x.shape
    dout = wa.shape[1]
    sd = jax.ShapeDtypeStruct((n, dout), jnp.float32)
    return pl.pallas_call(
        _mm2_body,
        grid=(n // MBLK,),
        in_specs=[pl.BlockSpec((MBLK, d), lambda i: (i, 0)),
                  pl.BlockSpec((d, dout), lambda i: (0, 0)),
                  pl.BlockSpec((d, dout), lambda i: (0, 0)),
                  pl.BlockSpec((1, dout), lambda i: (0, 0))],
        out_specs=[pl.BlockSpec((MBLK, dout), lambda i: (i, 0)),
                   pl.BlockSpec((MBLK, dout), lambda i: (i, 0))],
        out_shape=[sd, sd],
    )(x, wa, wb, bb.reshape(1, dout))


def _mm_body(x_ref, w_ref, b_ref, o_ref):
    o_ref[...] = jnp.dot(x_ref[...], w_ref[...],
                         preferred_element_type=jnp.float32) + b_ref[...]


def _matmul(x, w, b, mblk=MBLK):
    n, d = x.shape
    dout = w.shape[1]
    return pl.pallas_call(
        _mm_body,
        grid=(n // mblk,),
        in_specs=[pl.BlockSpec((mblk, d), lambda i: (i, 0)),
                  pl.BlockSpec((d, dout), lambda i: (0, 0)),
                  pl.BlockSpec((1, dout), lambda i: (0, 0))],
        out_specs=pl.BlockSpec((mblk, dout), lambda i: (i, 0)),
        out_shape=jax.ShapeDtypeStruct((n, dout), jnp.float32),
    )(x, w, b.reshape(1, dout))


def _tgcn_layer(x, src, dst, wq, bq, wk, bk, wv, bv, g, be, heads,
                z16, z128):
    q2d, k2d, v2d = _proj(x, wq, bq, wk, bk, wv, bv, heads)
    logits = _make_logits(heads)(q2d, k2d, src, dst)
    m = _gmax(logits, heads)
    denoms = _make_denom(heads)(logits, m, dst, z16)
    rden = _dencomb(denoms)
    opre = _make_msg(heads)(v2d, src, dst, logits, m, rden, z128)
    return _bn_relu(opre, g, be, 2 * heads)


def kernel(x, edge_index, Wq1, bq1, Wk1, bk1, Wv1, bv1, g1, be1,
           Wq2, bq2, Wk2, bk2, Wv2, bv2, g2, be2, Wm1, bm1, Wm2, bm2):
    src = edge_index[0]
    dst = edge_index[1]
    z16 = jnp.zeros((NPT, 16), jnp.float32)
    z128 = jnp.zeros((NPT, 128), jnp.float32)

    h1 = _tgcn_layer(x, src, dst, Wq1, bq1, Wk1, bk1, Wv1, bv1, g1, be1,
                     HEADS1, z16, z128)
    h2 = _tgcn_layer(h1, src, dst, Wq2, bq2, Wk2, bk2, Wv2, bv2, g2, be2,
                     1, z16, z128)

    a, b = _ab_matmul(h2, Wm1[:HID], Wm1[HID:], bm1)
    z = _make_edgemlp()(a, b, src, dst)
    return _matmul(z, Wm2, bm2, mblk=4000)
